# Initial kernel scaffold; baseline (speedup 1.0000x reference)
import jax
import jax.numpy as jnp
from jax.experimental import pallas as pl


def kernel(obs, params):
    raise NotImplementedError("write your pallas kernel here")


# speedup vs baseline: 1.0977x; 1.0977x over previous
import functools

import jax
import jax.numpy as jnp
from jax import lax
from jax.experimental import pallas as pl
from jax.experimental.pallas import tpu as pltpu
from jax.experimental.pallas import tpu_sc as plsc

_EMB_DIM = 16
_NE = 8192
_ROWS = 128 * 12 * 12
_RBLK = 1152
_CCHUNK = 4096
_NCHUNK = _NE // _CCHUNK


def _vq_argmin_kernel(lhs_ref, z2_ref, embT_ref, e2_ref, idx_ref):
    lhs = lhs_ref[...]
    z2 = z2_ref[...]

    acc_v = jnp.full((lhs.shape[0], 1), jnp.inf, jnp.float32)
    acc_i = jnp.zeros((lhs.shape[0], 1), jnp.int32)
    for c in range(_NCHUNK):
        m = lax.dot_general(lhs, embT_ref[c], (((1,), (0,)), ((), ())),
                            preferred_element_type=jnp.float32)
        d = (z2 + e2_ref[c]) - m
        tmin = jnp.min(d, axis=1, keepdims=True)
        iota = lax.broadcasted_iota(jnp.int32, d.shape, 1)
        cand = jnp.where(d == tmin, iota, jnp.int32(2**30))
        targ = jnp.min(cand, axis=1, keepdims=True) + c * _CCHUNK
        win = tmin < acc_v
        tmin_bf = tmin.astype(jnp.bfloat16).astype(jnp.float32)
        acc_v = jnp.where(win, tmin_bf, acc_v)
        acc_i = jnp.where(win, targ, acc_i)
    idx_ref[...] = acc_i


def _vq_argmin(lhs_bf, z2, embT3, e23):
    grid = (_ROWS // _RBLK,)
    return pl.pallas_call(
        _vq_argmin_kernel,
        grid=grid,
        in_specs=[
            pl.BlockSpec((_RBLK, _EMB_DIM), lambda i: (i, 0)),
            pl.BlockSpec((_RBLK, 1), lambda i: (i, 0)),
            pl.BlockSpec((_NCHUNK, _EMB_DIM, _CCHUNK), lambda i: (0, 0, 0)),
            pl.BlockSpec((_NCHUNK, 1, _CCHUNK), lambda i: (0, 0, 0)),
        ],
        out_specs=pl.BlockSpec((_RBLK, 1), lambda i: (i, 0)),
        out_shape=jax.ShapeDtypeStruct((_ROWS, 1), jnp.int32),
    )(lhs_bf, z2, embT3, e23)


def _make_sc_gather():
    info = plsc.get_sparse_core_info()
    nw = info.num_cores * info.num_subcores
    b_per_w = _ROWS // nw
    mesh = plsc.VectorSubcoreMesh(core_axis_name="c", subcore_axis_name="s")

    @functools.partial(
        pl.kernel, mesh=mesh,
        compiler_params=pltpu.CompilerParams(use_tc_tiling_on_sc=False),
        out_type=jax.ShapeDtypeStruct((_ROWS, _EMB_DIM), jnp.float32),
        scratch_types=[
            pltpu.VMEM((b_per_w,), jnp.int32),
            pltpu.VMEM((b_per_w, _EMB_DIM), jnp.float32),
            pltpu.SemaphoreType.DMA,
        ],
    )
    def gather(emb_hbm, idx_hbm, out_hbm, idx_v, rows_v, sem):
        wid = lax.axis_index("s") * info.num_cores + lax.axis_index("c")
        base = wid * b_per_w
        pltpu.sync_copy(idx_hbm.at[pl.ds(base, b_per_w)], idx_v)
        pltpu.async_copy(emb_hbm.at[idx_v], rows_v, sem).wait()
        pltpu.sync_copy(rows_v, out_hbm.at[pl.ds(base, b_per_w)])

    return gather


_sc_gather = None


def _conv(x, w, b, stride, pad):
    y = lax.conv_general_dilated(
        x, w, (stride, stride), [(pad, pad), (pad, pad)],
        dimension_numbers=('NCHW', 'OIHW', 'NCHW'))
    return y + b[None, :, None, None]


def kernel(obs, params):
    global _sc_gather
    x = jax.nn.relu(_conv(obs, params['W1'], params['b1'], 2, 1))
    x = jax.nn.relu(_conv(x, params['W2'], params['b2'], 2, 1))
    x = _conv(x, params['W3'], params['b3'], 1, 1)
    for r in params['res']:
        h = jax.nn.relu(x)
        h = _conv(h, r['Wa'], r['ba'], 1, 1)
        h = jax.nn.relu(h)
        h = _conv(h, r['Wb'], r['bb'], 1, 0)
        x = x + h
    x = jax.nn.relu(x)
    z = _conv(x, params['Wp'], params['bp'], 1, 0)
    zt = jnp.transpose(z, (0, 2, 3, 1))
    flat = zt.reshape(-1, _EMB_DIM)

    emb = params['emb']
    z2 = jnp.sum(flat ** 2, axis=1, keepdims=True)
    e2 = jnp.sum(emb ** 2, axis=1)
    lhs_bf = (2.0 * flat).astype(jnp.bfloat16)
    emb_bf = emb.astype(jnp.bfloat16)
    embT3 = emb_bf.T.reshape(_EMB_DIM, _NCHUNK, _CCHUNK).transpose(1, 0, 2)
    e23 = e2.reshape(_NCHUNK, 1, _CCHUNK)

    idx = _vq_argmin(lhs_bf, z2, embT3, e23)[:, 0]

    if _sc_gather is None:
        _sc_gather = _make_sc_gather()
    quant = _sc_gather(emb, idx)

    st = flat + (quant - flat)
    q = st.reshape(obs.shape[0], 12 * 12, _EMB_DIM).transpose(0, 2, 1)
    return q.reshape(obs.shape[0], -1)


# speedup vs baseline: 1.2509x; 1.1396x over previous
import functools

import jax
import jax.numpy as jnp
from jax import lax
from jax.experimental import pallas as pl
from jax.experimental.pallas import tpu as pltpu
from jax.experimental.pallas import tpu_sc as plsc

_EMB_DIM = 16
_NE = 8192
_ROWS = 128 * 12 * 12
_RBLK = 1152
_CCHUNK = 4096
_NCHUNK = _NE // _CCHUNK


def _vq_argmin_kernel(lhs_ref, z2_ref, embT_ref, e2_ref, idx_ref):
    lhs = lhs_ref[...]
    z2 = z2_ref[...]
    nrow = lhs.shape[0]
    SUB = 256
    nsub = _CCHUNK // SUB
    iota = lax.broadcasted_iota(jnp.int32, (nrow, SUB), 1)
    big = jnp.int32(2**30)

    acc_v = jnp.full((nrow, 1), jnp.inf, jnp.float32)
    acc_i = jnp.zeros((nrow, 1), jnp.int32)
    for c in range(_NCHUNK):
        run_v = jnp.full((nrow, SUB), jnp.inf, jnp.float32)
        run_i = jnp.zeros((nrow, SUB), jnp.int32)
        for g in range(nsub):
            m = lax.dot_general(
                lhs, embT_ref[c, :, g * SUB:(g + 1) * SUB],
                (((1,), (0,)), ((), ())),
                preferred_element_type=jnp.float32)
            d = (z2 + e2_ref[c, :, g * SUB:(g + 1) * SUB]) - m
            j = iota + (c * _CCHUNK + g * SUB)
            upd = d < run_v
            run_v = jnp.where(upd, d, run_v)
            run_i = jnp.where(upd, j, run_i)
        tmin = jnp.min(run_v, axis=1, keepdims=True)
        cand = jnp.where(run_v == tmin, run_i, big)
        targ = jnp.min(cand, axis=1, keepdims=True)
        win = tmin < acc_v
        tmin_bf = tmin.astype(jnp.bfloat16).astype(jnp.float32)
        acc_v = jnp.where(win, tmin_bf, acc_v)
        acc_i = jnp.where(win, targ, acc_i)
    idx_ref[...] = acc_i


def _vq_argmin(lhs_bf, z2, embT3, e23):
    grid = (_ROWS // _RBLK,)
    return pl.pallas_call(
        _vq_argmin_kernel,
        grid=grid,
        in_specs=[
            pl.BlockSpec((_RBLK, _EMB_DIM), lambda i: (i, 0)),
            pl.BlockSpec((_RBLK, 1), lambda i: (i, 0)),
            pl.BlockSpec((_NCHUNK, _EMB_DIM, _CCHUNK), lambda i: (0, 0, 0)),
            pl.BlockSpec((_NCHUNK, 1, _CCHUNK), lambda i: (0, 0, 0)),
        ],
        out_specs=pl.BlockSpec((_RBLK, 1), lambda i: (i, 0)),
        out_shape=jax.ShapeDtypeStruct((_ROWS, 1), jnp.int32),
    )(lhs_bf, z2, embT3, e23)


def _make_sc_gather():
    info = plsc.get_sparse_core_info()
    nw = info.num_cores * info.num_subcores
    b_per_w = _ROWS // nw
    mesh = plsc.VectorSubcoreMesh(core_axis_name="c", subcore_axis_name="s")

    @functools.partial(
        pl.kernel, mesh=mesh,
        compiler_params=pltpu.CompilerParams(use_tc_tiling_on_sc=False),
        out_type=jax.ShapeDtypeStruct((_ROWS, _EMB_DIM), jnp.float32),
        scratch_types=[
            pltpu.VMEM((b_per_w,), jnp.int32),
            pltpu.VMEM((b_per_w, _EMB_DIM), jnp.float32),
            pltpu.SemaphoreType.DMA,
        ],
    )
    def gather(emb_hbm, idx_hbm, out_hbm, idx_v, rows_v, sem):
        wid = lax.axis_index("s") * info.num_cores + lax.axis_index("c")
        base = wid * b_per_w
        pltpu.sync_copy(idx_hbm.at[pl.ds(base, b_per_w)], idx_v)
        pltpu.async_copy(emb_hbm.at[idx_v], rows_v, sem).wait()
        pltpu.sync_copy(rows_v, out_hbm.at[pl.ds(base, b_per_w)])

    return gather


_sc_gather = None


def _conv(x, w, b, stride, pad):
    y = lax.conv_general_dilated(
        x, w, (stride, stride), [(pad, pad), (pad, pad)],
        dimension_numbers=('NCHW', 'OIHW', 'NCHW'))
    return y + b[None, :, None, None]


def kernel(obs, params):
    global _sc_gather
    x = jax.nn.relu(_conv(obs, params['W1'], params['b1'], 2, 1))
    x = jax.nn.relu(_conv(x, params['W2'], params['b2'], 2, 1))
    x = _conv(x, params['W3'], params['b3'], 1, 1)
    for r in params['res']:
        h = jax.nn.relu(x)
        h = _conv(h, r['Wa'], r['ba'], 1, 1)
        h = jax.nn.relu(h)
        h = _conv(h, r['Wb'], r['bb'], 1, 0)
        x = x + h
    x = jax.nn.relu(x)
    z = _conv(x, params['Wp'], params['bp'], 1, 0)
    zt = jnp.transpose(z, (0, 2, 3, 1))
    flat = zt.reshape(-1, _EMB_DIM)

    emb = params['emb']
    z2 = jnp.sum(flat ** 2, axis=1, keepdims=True)
    e2 = jnp.sum(emb ** 2, axis=1)
    lhs_bf = (2.0 * flat).astype(jnp.bfloat16)
    emb_bf = emb.astype(jnp.bfloat16)
    embT3 = emb_bf.T.reshape(_EMB_DIM, _NCHUNK, _CCHUNK).transpose(1, 0, 2)
    e23 = e2.reshape(_NCHUNK, 1, _CCHUNK)

    idx = _vq_argmin(lhs_bf, z2, embT3, e23)[:, 0]

    if _sc_gather is None:
        _sc_gather = _make_sc_gather()
    quant = _sc_gather(emb, idx)

    st = flat + (quant - flat)
    q = st.reshape(obs.shape[0], 12 * 12, _EMB_DIM).transpose(0, 2, 1)
    return q.reshape(obs.shape[0], -1)


# speedup vs baseline: 1.2910x; 1.0320x over previous
import functools

import jax
import jax.numpy as jnp
from jax import lax
from jax.experimental import pallas as pl
from jax.experimental.pallas import tpu as pltpu
from jax.experimental.pallas import tpu_sc as plsc

_EMB_DIM = 16
_NE = 8192
_ROWS = 128 * 12 * 12
_RBLK = 1152
_CCHUNK = 4096
_NCHUNK = _NE // _CCHUNK


def _vq_argmin_kernel(lhs_ref, z2_ref, embT_ref, e2_ref, idx_ref):
    lhs = lhs_ref[...]
    z2 = z2_ref[...]
    nrow = lhs.shape[0]
    SUB = 256
    nsub = _CCHUNK // SUB
    iota = lax.broadcasted_iota(jnp.int32, (nrow, SUB), 1)
    big = jnp.int32(2**30)

    acc_v = jnp.full((nrow, 1), jnp.inf, jnp.float32)
    acc_i = jnp.zeros((nrow, 1), jnp.int32)
    for c in range(_NCHUNK):
        run_v = jnp.full((nrow, SUB), jnp.inf, jnp.float32)
        run_i = jnp.zeros((nrow, SUB), jnp.int32)
        for g in range(nsub):
            m = lax.dot_general(
                lhs, embT_ref[c, :, g * SUB:(g + 1) * SUB],
                (((1,), (0,)), ((), ())),
                preferred_element_type=jnp.float32)
            d = (z2 + e2_ref[c, :, g * SUB:(g + 1) * SUB]) - m
            j = iota + (c * _CCHUNK + g * SUB)
            upd = d < run_v
            run_v = jnp.where(upd, d, run_v)
            run_i = jnp.where(upd, j, run_i)
        tmin = jnp.min(run_v, axis=1, keepdims=True)
        cand = jnp.where(run_v == tmin, run_i, big)
        targ = jnp.min(cand, axis=1, keepdims=True)
        win = tmin < acc_v
        tmin_bf = tmin.astype(jnp.bfloat16).astype(jnp.float32)
        acc_v = jnp.where(win, tmin_bf, acc_v)
        acc_i = jnp.where(win, targ, acc_i)
    idx_ref[...] = acc_i


def _vq_argmin(lhs_bf, z2, embT3, e23):
    grid = (_ROWS // _RBLK,)
    return pl.pallas_call(
        _vq_argmin_kernel,
        grid=grid,
        in_specs=[
            pl.BlockSpec((_RBLK, _EMB_DIM), lambda i: (i, 0)),
            pl.BlockSpec((_RBLK, 1), lambda i: (i, 0)),
            pl.BlockSpec((_NCHUNK, _EMB_DIM, _CCHUNK), lambda i: (0, 0, 0)),
            pl.BlockSpec((_NCHUNK, 1, _CCHUNK), lambda i: (0, 0, 0)),
        ],
        out_specs=pl.BlockSpec((_RBLK, 1), lambda i: (i, 0)),
        out_shape=jax.ShapeDtypeStruct((_ROWS, 1), jnp.int32),
    )(lhs_bf, z2, embT3, e23)


def _make_sc_gather():
    info = plsc.get_sparse_core_info()
    nw = info.num_cores * info.num_subcores
    b_per_w = _ROWS // nw
    img_per_w = b_per_w // 144
    out_per_w = img_per_w * 16 * 144
    mesh = plsc.VectorSubcoreMesh(core_axis_name="c", subcore_axis_name="s")

    @functools.partial(
        pl.kernel, mesh=mesh,
        compiler_params=pltpu.CompilerParams(use_tc_tiling_on_sc=False,
                                             needs_layout_passes=False),
        out_type=jax.ShapeDtypeStruct((_ROWS * _EMB_DIM,), jnp.float32),
        scratch_types=[
            pltpu.VMEM((b_per_w,), jnp.int32),
            pltpu.VMEM((b_per_w, _EMB_DIM), jnp.float32),
            pltpu.VMEM((b_per_w, _EMB_DIM), jnp.float32),
            pltpu.VMEM((out_per_w,), jnp.float32),
            pltpu.SemaphoreType.DMA,
        ],
    )
    def gather(emb_hbm, idx_hbm, flat_hbm, out_hbm,
               idx_v, rows_v, flat_v, out_v, sem):
        wid = lax.axis_index("s") * info.num_cores + lax.axis_index("c")
        base = wid * b_per_w
        pltpu.sync_copy(idx_hbm.at[pl.ds(base, b_per_w)], idx_v)
        pltpu.sync_copy(flat_hbm.at[pl.ds(base, b_per_w)], flat_v)
        pltpu.async_copy(emb_hbm.at[idx_v], rows_v, sem).wait()
        lanes = lax.iota(jnp.int32, 16) * 144

        def body(i, _):
            f = flat_v[i, :]
            r = rows_v[i, :]
            st = f + (r - f)
            b_loc = i // 144
            p = i - b_loc * 144
            plsc.store_scatter(out_v, [lanes + (b_loc * 2304 + p)], st)
            return _

        lax.fori_loop(0, b_per_w, body, 0)
        pltpu.sync_copy(out_v, out_hbm.at[pl.ds(wid * out_per_w, out_per_w)])

    return gather


_sc_gather = None


def _conv(x, w, b, stride, pad):
    y = lax.conv_general_dilated(
        x, w, (stride, stride), [(pad, pad), (pad, pad)],
        dimension_numbers=('NCHW', 'OIHW', 'NCHW'))
    return y + b[None, :, None, None]


def kernel(obs, params):
    global _sc_gather
    x = jax.nn.relu(_conv(obs, params['W1'], params['b1'], 2, 1))
    x = jax.nn.relu(_conv(x, params['W2'], params['b2'], 2, 1))
    x = _conv(x, params['W3'], params['b3'], 1, 1)
    for r in params['res']:
        h = jax.nn.relu(x)
        h = _conv(h, r['Wa'], r['ba'], 1, 1)
        h = jax.nn.relu(h)
        h = _conv(h, r['Wb'], r['bb'], 1, 0)
        x = x + h
    x = jax.nn.relu(x)
    z = _conv(x, params['Wp'], params['bp'], 1, 0)
    zt = jnp.transpose(z, (0, 2, 3, 1))
    flat = zt.reshape(-1, _EMB_DIM)

    emb = params['emb']
    z2 = jnp.sum(flat ** 2, axis=1, keepdims=True)
    e2 = jnp.sum(emb ** 2, axis=1)
    lhs_bf = (2.0 * flat).astype(jnp.bfloat16)
    emb_bf = emb.astype(jnp.bfloat16)
    embT3 = emb_bf.T.reshape(_EMB_DIM, _NCHUNK, _CCHUNK).transpose(1, 0, 2)
    e23 = e2.reshape(_NCHUNK, 1, _CCHUNK)

    idx = _vq_argmin(lhs_bf, z2, embT3, e23)[:, 0]

    if _sc_gather is None:
        _sc_gather = _make_sc_gather()
    out = _sc_gather(emb, idx, flat)
    return out.reshape(obs.shape[0], -1)


# speedup vs baseline: 1.3021x; 1.0086x over previous
import functools

import jax
import jax.numpy as jnp
from jax import lax
from jax.experimental import pallas as pl
from jax.experimental.pallas import tpu as pltpu
from jax.experimental.pallas import tpu_sc as plsc

_EMB_DIM = 16
_NE = 8192
_ROWS = 128 * 12 * 12
_RBLK = 1152
_CCHUNK = 4096
_NCHUNK = _NE // _CCHUNK


def _vq_argmin_kernel(lhs_ref, z2_ref, embT_ref, e2_ref, idx_ref):
    lhs = lhs_ref[...]
    z2 = z2_ref[...]
    nrow = lhs.shape[0]
    SUB = 256
    nsub = _CCHUNK // SUB
    iota = lax.broadcasted_iota(jnp.int32, (nrow, SUB), 1)
    big = jnp.int32(2**30)

    acc_v = jnp.full((nrow, 1), jnp.inf, jnp.float32)
    acc_i = jnp.zeros((nrow, 1), jnp.int32)
    for c in range(_NCHUNK):
        run_v = jnp.full((nrow, SUB), jnp.inf, jnp.float32)
        run_i = jnp.zeros((nrow, SUB), jnp.int32)
        for g in range(nsub):
            m = lax.dot_general(
                lhs, embT_ref[c, :, g * SUB:(g + 1) * SUB],
                (((1,), (0,)), ((), ())),
                preferred_element_type=jnp.float32)
            d = (z2 + e2_ref[c, :, g * SUB:(g + 1) * SUB]) - m
            j = iota + (c * _CCHUNK + g * SUB)
            upd = d < run_v
            run_v = jnp.where(upd, d, run_v)
            run_i = jnp.where(upd, j, run_i)
        tmin = jnp.min(run_v, axis=1, keepdims=True)
        cand = jnp.where(run_v == tmin, run_i, big)
        targ = jnp.min(cand, axis=1, keepdims=True)
        win = tmin < acc_v
        tmin_bf = tmin.astype(jnp.bfloat16).astype(jnp.float32)
        acc_v = jnp.where(win, tmin_bf, acc_v)
        acc_i = jnp.where(win, targ, acc_i)
    idx_ref[...] = acc_i


def _vq_argmin(lhs_bf, z2, embT3, e23):
    grid = (_ROWS // _RBLK,)
    return pl.pallas_call(
        _vq_argmin_kernel,
        grid=grid,
        in_specs=[
            pl.BlockSpec((_RBLK, _EMB_DIM), lambda i: (i, 0)),
            pl.BlockSpec((_RBLK, 1), lambda i: (i, 0)),
            pl.BlockSpec((_NCHUNK, _EMB_DIM, _CCHUNK), lambda i: (0, 0, 0)),
            pl.BlockSpec((_NCHUNK, 1, _CCHUNK), lambda i: (0, 0, 0)),
        ],
        out_specs=pl.BlockSpec((_RBLK, 1), lambda i: (i, 0)),
        out_shape=jax.ShapeDtypeStruct((_ROWS, 1), jnp.int32),
    )(lhs_bf, z2, embT3, e23)


def _make_sc_gather():
    info = plsc.get_sparse_core_info()
    nw = info.num_cores * info.num_subcores
    b_per_w = _ROWS // nw
    img_per_w = b_per_w // 144
    out_per_w = img_per_w * 16 * 144
    mesh = plsc.VectorSubcoreMesh(core_axis_name="c", subcore_axis_name="s")

    @functools.partial(
        pl.kernel, mesh=mesh,
        compiler_params=pltpu.CompilerParams(use_tc_tiling_on_sc=False,
                                             needs_layout_passes=False),
        out_type=jax.ShapeDtypeStruct((_ROWS * _EMB_DIM,), jnp.float32),
        scratch_types=[
            pltpu.VMEM((b_per_w,), jnp.int32),
            pltpu.VMEM((b_per_w, _EMB_DIM), jnp.float32),
            pltpu.VMEM((b_per_w, _EMB_DIM), jnp.float32),
            pltpu.VMEM((out_per_w,), jnp.float32),
            pltpu.SemaphoreType.DMA,
        ],
    )
    def gather(emb_hbm, idx_hbm, flat_hbm, out_hbm,
               idx_v, rows_v, flat_v, out_v, sem):
        wid = lax.axis_index("s") * info.num_cores + lax.axis_index("c")
        base = wid * b_per_w
        pltpu.sync_copy(idx_hbm.at[pl.ds(base, b_per_w)], idx_v)
        pltpu.sync_copy(flat_hbm.at[pl.ds(base, b_per_w)], flat_v)
        pltpu.async_copy(emb_hbm.at[idx_v], rows_v, sem).wait()
        lanes = lax.iota(jnp.int32, 16) * 144

        @plsc.parallel_loop(0, b_per_w, unroll=8)
        def body(i):
            f = flat_v[i, :]
            r = rows_v[i, :]
            st = f + (r - f)
            b_loc = i // 144
            p = i - b_loc * 144
            plsc.store_scatter(out_v, [lanes + (b_loc * 2304 + p)], st)
        pltpu.sync_copy(out_v, out_hbm.at[pl.ds(wid * out_per_w, out_per_w)])

    return gather


_sc_gather = None


def _conv(x, w, b, stride, pad):
    y = lax.conv_general_dilated(
        x, w, (stride, stride), [(pad, pad), (pad, pad)],
        dimension_numbers=('NCHW', 'OIHW', 'NCHW'))
    return y + b[None, :, None, None]


def kernel(obs, params):
    global _sc_gather
    x = jax.nn.relu(_conv(obs, params['W1'], params['b1'], 2, 1))
    x = jax.nn.relu(_conv(x, params['W2'], params['b2'], 2, 1))
    x = _conv(x, params['W3'], params['b3'], 1, 1)
    for r in params['res']:
        h = jax.nn.relu(x)
        h = _conv(h, r['Wa'], r['ba'], 1, 1)
        h = jax.nn.relu(h)
        h = _conv(h, r['Wb'], r['bb'], 1, 0)
        x = x + h
    x = jax.nn.relu(x)
    z = _conv(x, params['Wp'], params['bp'], 1, 0)
    zt = jnp.transpose(z, (0, 2, 3, 1))
    flat = zt.reshape(-1, _EMB_DIM)

    emb = params['emb']
    z2 = jnp.sum(flat ** 2, axis=1, keepdims=True)
    e2 = jnp.sum(emb ** 2, axis=1)
    lhs_bf = (2.0 * flat).astype(jnp.bfloat16)
    emb_bf = emb.astype(jnp.bfloat16)
    embT3 = emb_bf.T.reshape(_EMB_DIM, _NCHUNK, _CCHUNK).transpose(1, 0, 2)
    e23 = e2.reshape(_NCHUNK, 1, _CCHUNK)

    idx = _vq_argmin(lhs_bf, z2, embT3, e23)[:, 0]

    if _sc_gather is None:
        _sc_gather = _make_sc_gather()
    out = _sc_gather(emb, idx, flat)
    return out.reshape(obs.shape[0], -1)
